# baseline (speedup 1.0000x reference)
import functools

import jax
import jax.numpy as jnp
from jax import lax
from jax.experimental import pallas as pl
from jax.experimental.pallas import tpu as pltpu
from jax.experimental.pallas import tpu_sc as plsc

E = 16
D = 512
F = 1024
TM = 256
EPS = 1e-12


def _sc_permute(table, src_idx, dst_idx, n_out):
    n, d = src_idx.shape[0], table.shape[1]
    info = plsc.get_sparse_core_info()
    nw = info.num_cores * info.num_subcores
    per_w = n // nw
    ch = max(c for c in range(8, 81, 8) if per_w % c == 0)
    n_ch = per_w // ch
    nb = min(3, n_ch)
    src3 = src_idx.reshape(nw, n_ch, ch)
    dst3 = dst_idx.reshape(nw, n_ch, ch)
    mesh = plsc.VectorSubcoreMesh(core_axis_name="c", subcore_axis_name="s")

    @functools.partial(
        pl.kernel,
        mesh=mesh,
        out_type=jax.ShapeDtypeStruct((n_out, d), table.dtype),
        scratch_types=[
            pltpu.VMEM((n_ch, ch), jnp.int32),
            pltpu.VMEM((n_ch, ch), jnp.int32),
            pltpu.VMEM((nb, ch, d), table.dtype),
            pltpu.SemaphoreType.DMA((nb,)),
            pltpu.SemaphoreType.DMA((nb,)),
        ],
    )
    def permute_k(table_hbm, src_hbm, dst_hbm, out_hbm, src_v, dst_v, rows_v,
                  gsem, ssem):
        wid = lax.axis_index("s") * info.num_cores + lax.axis_index("c")
        pltpu.sync_copy(src_hbm.at[wid], src_v)
        pltpu.sync_copy(dst_hbm.at[wid], dst_v)

        def start_gather(i):
            return pltpu.async_copy(
                table_hbm.at[src_v.at[i]], rows_v.at[i % nb], gsem.at[i % nb])

        def start_scatter(i):
            return pltpu.async_copy(
                rows_v.at[i % nb], out_hbm.at[dst_v.at[i]], ssem.at[i % nb])

        copies = [None] * n_ch
        stores = [None] * n_ch
        for i in range(min(nb - 1, n_ch)):
            copies[i] = start_gather(i)
        for i in range(n_ch):
            copies[i].wait()
            stores[i] = start_scatter(i)
            j = i + nb - 1
            if j < n_ch:
                if i >= 1:
                    stores[i - 1].wait()
                copies[j] = start_gather(j)
        for i in range(max(0, n_ch - nb), n_ch):
            stores[i].wait()

    return permute_k(table, src3, dst3)


def _mlp_body(meta_ref, x_ref, w1_ref, b1_ref, w2_ref, b2_ref, gm_ref, bt_ref,
              o_ref):
    j = pl.program_id(0)
    lo = meta_ref[2, j]
    hi = meta_ref[3, j]
    x = x_ref[...]
    h = lax.dot_general(x, w1_ref[0], (((1,), (1,)), ((), ())),
                        preferred_element_type=jnp.float32)
    h = jax.nn.gelu(h + b1_ref[0])
    y = lax.dot_general(h, w2_ref[0], (((1,), (1,)), ((), ())),
                        preferred_element_type=jnp.float32)
    z = y + b2_ref[0] + x
    mu = jnp.mean(z, axis=1, keepdims=True)
    zc = z - mu
    var = jnp.mean(zc * zc, axis=1, keepdims=True)
    zn = zc * lax.rsqrt(var + EPS)
    res = zn * gm_ref[...] + bt_ref[...]
    rows = lax.broadcasted_iota(jnp.int32, (TM, 1), 0)
    mask = (rows >= lo) & (rows < hi)
    o_ref[...] = jnp.where(mask, res, o_ref[...])


def kernel(hidden_states, route, W1, b1, W2, b2, gamma, beta):
    b, s, _ = hidden_states.shape
    t = b * s
    nt = t // TM
    ni = nt + E - 1

    x = hidden_states.reshape(t, D)
    r = route.astype(jnp.int32)
    iota_t = jnp.arange(t, dtype=jnp.int32)

    rsort, tok = lax.sort_key_val(r, iota_t)
    o = jnp.searchsorted(rsort, jnp.arange(E, dtype=jnp.int32)).astype(
        jnp.int32)
    counts = jnp.diff(jnp.append(o, jnp.int32(t)))
    first = o // TM
    last = jnp.where(counts > 0, (o + counts - 1) // TM, first)
    n_e = jnp.where(counts > 0, last - first + 1, 0)
    cum = jnp.append(jnp.int32(0), jnp.cumsum(n_e)).astype(jnp.int32)
    jidx = jnp.arange(ni, dtype=jnp.int32)
    e_of = jnp.clip(jnp.searchsorted(cum, jidx, side="right") - 1,
                    0, E - 1).astype(jnp.int32)
    tile_of = jnp.clip(first[e_of] + (jidx - cum[e_of]), 0, nt - 1)
    valid = jidx < cum[E]
    row0 = tile_of * TM
    lo = jnp.where(valid, jnp.clip(o[e_of] - row0, 0, TM), 0)
    hi = jnp.where(valid, jnp.clip(o[e_of] + counts[e_of] - row0, 0, TM), 0)
    meta = jnp.stack([tile_of, e_of, lo, hi]).astype(jnp.int32)

    x_s = _sc_permute(x, tok, iota_t, t)

    grid_spec = pltpu.PrefetchScalarGridSpec(
        num_scalar_prefetch=1,
        grid=(ni,),
        in_specs=[
            pl.BlockSpec((TM, D), lambda j, m: (m[0, j], 0)),
            pl.BlockSpec((1, F, D), lambda j, m: (m[1, j], 0, 0)),
            pl.BlockSpec((1, 1, F), lambda j, m: (m[1, j], 0, 0)),
            pl.BlockSpec((1, D, F), lambda j, m: (m[1, j], 0, 0)),
            pl.BlockSpec((1, 1, D), lambda j, m: (m[1, j], 0, 0)),
            pl.BlockSpec((1, D), lambda j, m: (0, 0)),
            pl.BlockSpec((1, D), lambda j, m: (0, 0)),
        ],
        out_specs=pl.BlockSpec((TM, D), lambda j, m: (m[0, j], 0)),
    )
    out_s = pl.pallas_call(
        _mlp_body,
        grid_spec=grid_spec,
        out_shape=jax.ShapeDtypeStruct((t, D), jnp.float32),
    )(meta, x_s, W1, b1.reshape(E, 1, F), W2, b2.reshape(E, 1, D),
      gamma.reshape(1, D), beta.reshape(1, D))

    y = _sc_permute(out_s, iota_t, tok, t)
    return y.reshape(b, s, D)


# speedup vs baseline: 1.0747x; 1.0747x over previous
import functools

import jax
import jax.numpy as jnp
from jax import lax
from jax.experimental import pallas as pl
from jax.experimental.pallas import tpu as pltpu
from jax.experimental.pallas import tpu_sc as plsc

E = 16
D = 512
F = 1024
TM = 512
EPS = 1e-12


def _sc_permute(table, idx, gather_side):
    n, d = idx.shape[0], table.shape[1]
    info = plsc.get_sparse_core_info()
    nw = info.num_cores * info.num_subcores
    per_w = n // nw
    ch = max(c for c in range(8, 81, 8) if per_w % c == 0)
    n_ch = per_w // ch
    nb = min(3, n_ch)
    idx3 = idx.reshape(nw, n_ch, ch)
    mesh = plsc.VectorSubcoreMesh(core_axis_name="c", subcore_axis_name="s")

    @functools.partial(
        pl.kernel,
        mesh=mesh,
        out_type=jax.ShapeDtypeStruct((n, d), table.dtype),
        scratch_types=[
            pltpu.VMEM((n_ch, ch), jnp.int32),
            pltpu.VMEM((nb, ch, d), table.dtype),
            pltpu.SemaphoreType.DMA((nb,)),
            pltpu.SemaphoreType.DMA((nb,)),
        ],
    )
    def permute_k(table_hbm, idx_hbm, out_hbm, idx_v, rows_v, gsem, ssem):
        wid = lax.axis_index("s") * info.num_cores + lax.axis_index("c")
        base = wid * per_w
        pltpu.sync_copy(idx_hbm.at[wid], idx_v)

        def start_read(i):
            src = (table_hbm.at[idx_v.at[i]] if gather_side
                   else table_hbm.at[pl.ds(base + i * ch, ch)])
            return pltpu.async_copy(src, rows_v.at[i % nb], gsem.at[i % nb])

        def start_write(i):
            dst = (out_hbm.at[pl.ds(base + i * ch, ch)] if gather_side
                   else out_hbm.at[idx_v.at[i]])
            return pltpu.async_copy(rows_v.at[i % nb], dst, ssem.at[i % nb])

        copies = [None] * n_ch
        stores = [None] * n_ch
        for i in range(min(nb - 1, n_ch)):
            copies[i] = start_read(i)
        for i in range(n_ch):
            copies[i].wait()
            stores[i] = start_write(i)
            j = i + nb - 1
            if j < n_ch:
                if i >= 1:
                    stores[i - 1].wait()
                copies[j] = start_read(j)
        for i in range(max(0, n_ch - nb), n_ch):
            stores[i].wait()

    return permute_k(table, idx3)


def _mlp_body(meta_ref, x_ref, w1_ref, b1_ref, w2_ref, b2_ref, gm_ref, bt_ref,
              o_ref):
    j = pl.program_id(0)
    lo = meta_ref[2, j]
    hi = meta_ref[3, j]
    x = x_ref[...]
    h = lax.dot_general(x, w1_ref[0], (((1,), (1,)), ((), ())),
                        preferred_element_type=jnp.float32)
    h = jax.nn.gelu(h + b1_ref[0])
    y = lax.dot_general(h, w2_ref[0], (((1,), (1,)), ((), ())),
                        preferred_element_type=jnp.float32)
    z = y + b2_ref[0] + x
    mu = jnp.mean(z, axis=1, keepdims=True)
    zc = z - mu
    var = jnp.mean(zc * zc, axis=1, keepdims=True)
    zn = zc * lax.rsqrt(var + EPS)
    res = zn * gm_ref[...] + bt_ref[...]
    rows = lax.broadcasted_iota(jnp.int32, (TM, 1), 0)
    mask = (rows >= lo) & (rows < hi)
    o_ref[...] = jnp.where(mask, res, o_ref[...])


def kernel(hidden_states, route, W1, b1, W2, b2, gamma, beta):
    b, s, _ = hidden_states.shape
    t = b * s
    nt = t // TM
    ni = nt + E - 1

    x = hidden_states.reshape(t, D)
    r = route.astype(jnp.int32)
    iota_t = jnp.arange(t, dtype=jnp.int32)

    rsort, tok = lax.sort_key_val(r, iota_t)
    o = jnp.searchsorted(rsort, jnp.arange(E, dtype=jnp.int32)).astype(
        jnp.int32)
    counts = jnp.diff(jnp.append(o, jnp.int32(t)))
    first = o // TM
    last = jnp.where(counts > 0, (o + counts - 1) // TM, first)
    n_e = jnp.where(counts > 0, last - first + 1, 0)
    cum = jnp.append(jnp.int32(0), jnp.cumsum(n_e)).astype(jnp.int32)
    jidx = jnp.arange(ni, dtype=jnp.int32)
    e_of = jnp.clip(jnp.searchsorted(cum, jidx, side="right") - 1,
                    0, E - 1).astype(jnp.int32)
    tile_of = jnp.clip(first[e_of] + (jidx - cum[e_of]), 0, nt - 1)
    valid = jidx < cum[E]
    row0 = tile_of * TM
    lo = jnp.where(valid, jnp.clip(o[e_of] - row0, 0, TM), 0)
    hi = jnp.where(valid, jnp.clip(o[e_of] + counts[e_of] - row0, 0, TM), 0)
    meta = jnp.stack([tile_of, e_of, lo, hi]).astype(jnp.int32)

    x_s = _sc_permute(x, tok, gather_side=True)

    grid_spec = pltpu.PrefetchScalarGridSpec(
        num_scalar_prefetch=1,
        grid=(ni,),
        in_specs=[
            pl.BlockSpec((TM, D), lambda j, m: (m[0, j], 0)),
            pl.BlockSpec((1, F, D), lambda j, m: (m[1, j], 0, 0)),
            pl.BlockSpec((1, 1, F), lambda j, m: (m[1, j], 0, 0)),
            pl.BlockSpec((1, D, F), lambda j, m: (m[1, j], 0, 0)),
            pl.BlockSpec((1, 1, D), lambda j, m: (m[1, j], 0, 0)),
            pl.BlockSpec((1, D), lambda j, m: (0, 0)),
            pl.BlockSpec((1, D), lambda j, m: (0, 0)),
        ],
        out_specs=pl.BlockSpec((TM, D), lambda j, m: (m[0, j], 0)),
    )
    out_s = pl.pallas_call(
        _mlp_body,
        grid_spec=grid_spec,
        out_shape=jax.ShapeDtypeStruct((t, D), jnp.float32),
    )(meta, x_s, W1, b1.reshape(E, 1, F), W2, b2.reshape(E, 1, D),
      gamma.reshape(1, D), beta.reshape(1, D))

    y = _sc_permute(out_s, tok, gather_side=False)
    return y.reshape(b, s, D)
